# Initial kernel scaffold; baseline (speedup 1.0000x reference)
import jax
import jax.numpy as jnp
from jax.experimental import pallas as pl


def kernel(adj, x, W1, b1, W2, b2):
    raise NotImplementedError("write your pallas kernel here")


# speedup vs baseline: 1.0309x; 1.0309x over previous
import functools

import jax
import jax.numpy as jnp
from jax.experimental import pallas as pl


def _gcn_layer_body(adj_ref, v_ref, w_ref, b_ref, o_ref, *, relu: bool):
    t = jnp.dot(adj_ref[...], v_ref[...], preferred_element_type=jnp.float32)
    o = jnp.dot(t, w_ref[...], preferred_element_type=jnp.float32) + b_ref[...]
    if relu:
        o = jnp.maximum(o, 0.0)
    o_ref[...] = o


def _gcn_layer(adj, v, w, b, *, relu: bool, block_rows: int):
    n, k = adj.shape
    d_in = v.shape[1]
    d_out = w.shape[1]
    if n % block_rows:
        block_rows = n
    grid = (n // block_rows,)
    return pl.pallas_call(
        functools.partial(_gcn_layer_body, relu=relu),
        grid=grid,
        in_specs=[
            pl.BlockSpec((block_rows, k), lambda i: (i, 0)),
            pl.BlockSpec((k, d_in), lambda i: (0, 0)),
            pl.BlockSpec((d_in, d_out), lambda i: (0, 0)),
            pl.BlockSpec((1, d_out), lambda i: (0, 0)),
        ],
        out_specs=pl.BlockSpec((block_rows, d_out), lambda i: (i, 0)),
        out_shape=jax.ShapeDtypeStruct((n, d_out), jnp.float32),
    )(adj, v, w, b)


def kernel(adj, x, W1, b1, W2, b2):
    b1 = b1.reshape(1, -1)
    b2 = b2.reshape(1, -1)
    h = _gcn_layer(adj, x, W1, b1, relu=True, block_rows=400)
    out = _gcn_layer(adj, h, W2, b2, relu=False, block_rows=400)
    return out
